# baseline (speedup 1.0000x reference)
import functools

import jax
import jax.numpy as jnp
from jax import lax
from jax.experimental import pallas as pl
from jax.experimental.pallas import tpu as pltpu
from jax.experimental.pallas import tpu_sc as plsc


def _make_gather(n_nodes: int, n_edges: int, d: int):
    info = plsc.get_sparse_core_info()
    nc, ns = info.num_cores, info.num_subcores
    nw = nc * ns
    assert n_edges % nw == 0
    b_per_w = n_edges // nw
    chunk = 48
    nbuf = 6
    lag = 2
    n_chunks = b_per_w // chunk
    tail = b_per_w - n_chunks * chunk
    pre_rows = 104
    assert chunk % 8 == 0 and tail % 8 == 0 and b_per_w % 8 == 0

    mesh = plsc.VectorSubcoreMesh(core_axis_name="c", subcore_axis_name="s")

    @functools.partial(
        pl.kernel,
        mesh=mesh,
        out_type=jax.ShapeDtypeStruct((n_edges, d), jnp.float32),
        scratch_types=[
            pltpu.VMEM((b_per_w,), jnp.int32),
            pltpu.VMEM((nbuf * chunk, d), jnp.float32),
            pltpu.VMEM((tail, d), jnp.float32),
            pltpu.VMEM_SHARED((n_nodes, d), jnp.float32),
            pltpu.SemaphoreType.DMA((nbuf,)),
            pltpu.SemaphoreType.DMA((nbuf,)),
            pltpu.SemaphoreType.DMA,
        ],
    )
    def gather_kernel(
        px_hbm, idx_hbm, out_hbm, idx_v, rows_v, tail_v, px_sh, gsem, ssem, tsem
    ):
        sid = lax.axis_index("s")
        wid = sid * nc + lax.axis_index("c")
        base = wid * b_per_w
        pltpu.async_copy(idx_hbm.at[pl.ds(base, b_per_w)], idx_v, tsem)

        rows_per_tile = 624
        n_pre = rows_per_tile // pre_rows
        for k in range(n_pre):
            off = sid * rows_per_tile + k * pre_rows
            pltpu.async_copy(
                px_hbm.at[pl.ds(off, pre_rows)], px_sh.at[pl.ds(off, pre_rows)], gsem.at[0]
            )

        @pl.when(sid == 0)
        def _stage_rest():
            off = ns * rows_per_tile
            rest = n_nodes - ns * rows_per_tile
            pltpu.sync_copy(px_hbm.at[pl.ds(off, rest)], px_sh.at[pl.ds(off, rest)])

        for k in range(n_pre):
            pltpu.make_async_copy(
                px_hbm.at[pl.ds(0, pre_rows)], px_sh.at[pl.ds(0, pre_rows)], gsem.at[0]
            ).wait()
        pltpu.make_async_copy(idx_hbm.at[pl.ds(0, b_per_w)], idx_v, tsem).wait()

        plsc.subcore_barrier()

        pltpu.async_copy(
            px_sh.at[idx_v.at[pl.ds(n_chunks * chunk, tail)]], tail_v, tsem
        )

        def start_gather(ci, b):
            idx_slice = idx_v.at[pl.ds(ci * chunk, chunk)]
            dst = rows_v.at[pl.ds(b * chunk, chunk)]
            if isinstance(ci, int):
                if ci % 2 == 0:
                    pltpu.async_copy(px_sh.at[idx_slice], dst, gsem.at[b])
                else:
                    pltpu.async_copy(px_hbm.at[idx_slice], dst, gsem.at[b])
            else:

                @pl.when(lax.rem(ci, 2) == 0)
                def _from_shared():
                    pltpu.async_copy(px_sh.at[idx_slice], dst, gsem.at[b])

                @pl.when(lax.rem(ci, 2) == 1)
                def _from_hbm():
                    pltpu.async_copy(px_hbm.at[idx_slice], dst, gsem.at[b])

        def wait_gather(b):
            pltpu.make_async_copy(
                px_sh.at[pl.ds(0, chunk)], rows_v.at[pl.ds(b * chunk, chunk)], gsem.at[b]
            ).wait()

        def start_store(ci, b):
            pltpu.async_copy(
                rows_v.at[pl.ds(b * chunk, chunk)],
                out_hbm.at[pl.ds(base + ci * chunk, chunk)],
                ssem.at[b],
            )

        def wait_store(b):
            pltpu.make_async_copy(
                rows_v.at[pl.ds(b * chunk, chunk)],
                out_hbm.at[pl.ds(0, chunk)],
                ssem.at[b],
            ).wait()

        for i in range(lag):
            start_gather(i, i)
        for i in range(lag, nbuf):
            start_gather(i, i)
            wait_gather(i - lag)
            start_store(i - lag, i - lag)

        def step(i, carry):
            bg = lax.rem(i, nbuf)
            wait_store(bg)
            start_gather(i, bg)
            j = i - lag
            bj = lax.rem(j, nbuf)
            wait_gather(bj)
            start_store(j, bj)
            return carry

        lax.fori_loop(nbuf, n_chunks, step, 0)

        for j in range(n_chunks - lag, n_chunks):
            wait_gather(j % nbuf)
            start_store(j, j % nbuf)

        pltpu.make_async_copy(px_sh.at[pl.ds(0, tail)], tail_v, tsem).wait()
        pltpu.async_copy(
            tail_v, out_hbm.at[pl.ds(base + n_chunks * chunk, tail)], tsem
        )

        for b in range(nbuf):
            wait_store(b)
        pltpu.make_async_copy(tail_v, out_hbm.at[pl.ds(0, tail)], tsem).wait()

    return gather_kernel


def kernel(px, pair_i, pair_j, W):
    del pair_i, W
    n_nodes, d = px.shape
    (n_edges,) = pair_j.shape
    fn = _make_gather(n_nodes, n_edges, d)
    return fn(px, pair_j.astype(jnp.int32))


# speedup vs baseline: 1.3629x; 1.3629x over previous
import functools

import jax
import jax.numpy as jnp
from jax import lax
from jax.experimental import pallas as pl
from jax.experimental.pallas import tpu as pltpu
from jax.experimental.pallas import tpu_sc as plsc


def _make_gather(n_nodes: int, n_edges: int, d: int):
    info = plsc.get_sparse_core_info()
    nc, ns = info.num_cores, info.num_subcores
    nw = nc * ns
    assert n_edges % nw == 0
    b_per_w = n_edges // nw
    chunk = 48
    nbuf = 6
    lag = 3
    n_chunks = b_per_w // chunk
    tail = b_per_w - n_chunks * chunk
    pre_rows = 104
    assert chunk % 8 == 0 and tail % 8 == 0 and b_per_w % 8 == 0

    mesh = plsc.VectorSubcoreMesh(core_axis_name="c", subcore_axis_name="s")

    @functools.partial(
        pl.kernel,
        mesh=mesh,
        out_type=jax.ShapeDtypeStruct((n_edges, d), jnp.float32),
        scratch_types=[
            pltpu.VMEM((b_per_w,), jnp.int32),
            pltpu.VMEM((nbuf * chunk, d), jnp.float32),
            pltpu.VMEM((tail, d), jnp.float32),
            pltpu.VMEM_SHARED((n_nodes, d), jnp.float32),
            pltpu.SemaphoreType.DMA((nbuf,)),
            pltpu.SemaphoreType.DMA((nbuf,)),
            pltpu.SemaphoreType.DMA,
        ],
    )
    def gather_kernel(
        px_hbm, idx_hbm, out_hbm, idx_v, rows_v, tail_v, px_sh, gsem, ssem, tsem
    ):
        sid = lax.axis_index("s")
        wid = sid * nc + lax.axis_index("c")
        base = wid * b_per_w
        pltpu.async_copy(idx_hbm.at[pl.ds(base, b_per_w)], idx_v, tsem)

        rows_per_tile = 624
        n_pre = rows_per_tile // pre_rows
        for k in range(n_pre):
            off = sid * rows_per_tile + k * pre_rows
            pltpu.async_copy(
                px_hbm.at[pl.ds(off, pre_rows)], px_sh.at[pl.ds(off, pre_rows)], gsem.at[0]
            )

        @pl.when(sid == 0)
        def _stage_rest():
            off = ns * rows_per_tile
            rest = n_nodes - ns * rows_per_tile
            pltpu.sync_copy(px_hbm.at[pl.ds(off, rest)], px_sh.at[pl.ds(off, rest)])

        for k in range(n_pre):
            pltpu.make_async_copy(
                px_hbm.at[pl.ds(0, pre_rows)], px_sh.at[pl.ds(0, pre_rows)], gsem.at[0]
            ).wait()
        pltpu.make_async_copy(idx_hbm.at[pl.ds(0, b_per_w)], idx_v, tsem).wait()

        plsc.subcore_barrier()

        pltpu.async_copy(
            px_sh.at[idx_v.at[pl.ds(n_chunks * chunk, tail)]], tail_v, tsem
        )

        def start_gather(ci, b):
            idx_slice = idx_v.at[pl.ds(ci * chunk, chunk)]
            pltpu.async_copy(
                px_sh.at[idx_slice], rows_v.at[pl.ds(b * chunk, chunk)], gsem.at[b]
            )

        def wait_gather(b):
            pltpu.make_async_copy(
                px_sh.at[pl.ds(0, chunk)], rows_v.at[pl.ds(b * chunk, chunk)], gsem.at[b]
            ).wait()

        def start_store(ci, b):
            pltpu.async_copy(
                rows_v.at[pl.ds(b * chunk, chunk)],
                out_hbm.at[pl.ds(base + ci * chunk, chunk)],
                ssem.at[b],
            )

        def wait_store(b):
            pltpu.make_async_copy(
                rows_v.at[pl.ds(b * chunk, chunk)],
                out_hbm.at[pl.ds(0, chunk)],
                ssem.at[b],
            ).wait()

        for i in range(lag):
            start_gather(i, i)
        for i in range(lag, nbuf):
            start_gather(i, i)
            wait_gather(i - lag)
            start_store(i - lag, i - lag)

        def step(i, carry):
            bg = lax.rem(i, nbuf)
            wait_store(bg)
            start_gather(i, bg)
            j = i - lag
            bj = lax.rem(j, nbuf)
            wait_gather(bj)
            start_store(j, bj)
            return carry

        lax.fori_loop(nbuf, n_chunks, step, 0)

        for j in range(n_chunks - lag, n_chunks):
            wait_gather(j % nbuf)
            start_store(j, j % nbuf)

        pltpu.make_async_copy(px_sh.at[pl.ds(0, tail)], tail_v, tsem).wait()
        pltpu.async_copy(
            tail_v, out_hbm.at[pl.ds(base + n_chunks * chunk, tail)], tsem
        )

        for b in range(nbuf):
            wait_store(b)
        pltpu.make_async_copy(tail_v, out_hbm.at[pl.ds(0, tail)], tsem).wait()

    return gather_kernel


def kernel(px, pair_i, pair_j, W):
    del pair_i, W
    n_nodes, d = px.shape
    (n_edges,) = pair_j.shape
    fn = _make_gather(n_nodes, n_edges, d)
    return fn(px, pair_j.astype(jnp.int32))


# speedup vs baseline: 1.3652x; 1.0016x over previous
import functools

import jax
import jax.numpy as jnp
from jax import lax
from jax.experimental import pallas as pl
from jax.experimental.pallas import tpu as pltpu
from jax.experimental.pallas import tpu_sc as plsc


def _make_gather(n_nodes: int, n_edges: int, d: int):
    info = plsc.get_sparse_core_info()
    nc, ns = info.num_cores, info.num_subcores
    nw = nc * ns
    assert n_edges % nw == 0
    b_per_w = n_edges // nw
    chunk = 24
    nbuf = 12
    lag = 5
    n_chunks = b_per_w // chunk
    tail = b_per_w - n_chunks * chunk
    pre_rows = 104
    assert chunk % 8 == 0 and tail % 8 == 0 and b_per_w % 8 == 0

    mesh = plsc.VectorSubcoreMesh(core_axis_name="c", subcore_axis_name="s")

    @functools.partial(
        pl.kernel,
        mesh=mesh,
        out_type=jax.ShapeDtypeStruct((n_edges, d), jnp.float32),
        scratch_types=[
            pltpu.VMEM((b_per_w,), jnp.int32),
            pltpu.VMEM((nbuf * chunk, d), jnp.float32),
            pltpu.VMEM((tail, d), jnp.float32),
            pltpu.VMEM_SHARED((n_nodes, d), jnp.float32),
            pltpu.SemaphoreType.DMA((nbuf,)),
            pltpu.SemaphoreType.DMA((nbuf,)),
            pltpu.SemaphoreType.DMA,
        ],
    )
    def gather_kernel(
        px_hbm, idx_hbm, out_hbm, idx_v, rows_v, tail_v, px_sh, gsem, ssem, tsem
    ):
        sid = lax.axis_index("s")
        wid = sid * nc + lax.axis_index("c")
        base = wid * b_per_w
        pltpu.async_copy(idx_hbm.at[pl.ds(base, b_per_w)], idx_v, tsem)

        rows_per_tile = 624
        n_pre = rows_per_tile // pre_rows
        for k in range(n_pre):
            off = sid * rows_per_tile + k * pre_rows
            pltpu.async_copy(
                px_hbm.at[pl.ds(off, pre_rows)], px_sh.at[pl.ds(off, pre_rows)], gsem.at[0]
            )

        @pl.when(sid == 0)
        def _stage_rest():
            off = ns * rows_per_tile
            rest = n_nodes - ns * rows_per_tile
            pltpu.sync_copy(px_hbm.at[pl.ds(off, rest)], px_sh.at[pl.ds(off, rest)])

        for k in range(n_pre):
            pltpu.make_async_copy(
                px_hbm.at[pl.ds(0, pre_rows)], px_sh.at[pl.ds(0, pre_rows)], gsem.at[0]
            ).wait()
        pltpu.make_async_copy(idx_hbm.at[pl.ds(0, b_per_w)], idx_v, tsem).wait()

        plsc.subcore_barrier()

        pltpu.async_copy(
            px_sh.at[idx_v.at[pl.ds(n_chunks * chunk, tail)]], tail_v, tsem
        )

        def start_gather(ci, b):
            idx_slice = idx_v.at[pl.ds(ci * chunk, chunk)]
            pltpu.async_copy(
                px_sh.at[idx_slice], rows_v.at[pl.ds(b * chunk, chunk)], gsem.at[b]
            )

        def wait_gather(b):
            pltpu.make_async_copy(
                px_sh.at[pl.ds(0, chunk)], rows_v.at[pl.ds(b * chunk, chunk)], gsem.at[b]
            ).wait()

        def start_store(ci, b):
            pltpu.async_copy(
                rows_v.at[pl.ds(b * chunk, chunk)],
                out_hbm.at[pl.ds(base + ci * chunk, chunk)],
                ssem.at[b],
            )

        def wait_store(b):
            pltpu.make_async_copy(
                rows_v.at[pl.ds(b * chunk, chunk)],
                out_hbm.at[pl.ds(0, chunk)],
                ssem.at[b],
            ).wait()

        for i in range(lag):
            start_gather(i, i)
        for i in range(lag, nbuf):
            start_gather(i, i)
            wait_gather(i - lag)
            start_store(i - lag, i - lag)

        def step(i, carry):
            bg = lax.rem(i, nbuf)
            wait_store(bg)
            start_gather(i, bg)
            j = i - lag
            bj = lax.rem(j, nbuf)
            wait_gather(bj)
            start_store(j, bj)
            return carry

        lax.fori_loop(nbuf, n_chunks, step, 0)

        for j in range(n_chunks - lag, n_chunks):
            wait_gather(j % nbuf)
            start_store(j, j % nbuf)

        pltpu.make_async_copy(px_sh.at[pl.ds(0, tail)], tail_v, tsem).wait()
        pltpu.async_copy(
            tail_v, out_hbm.at[pl.ds(base + n_chunks * chunk, tail)], tsem
        )

        for b in range(nbuf):
            wait_store(b)
        pltpu.make_async_copy(tail_v, out_hbm.at[pl.ds(0, tail)], tsem).wait()

    return gather_kernel


def kernel(px, pair_i, pair_j, W):
    del pair_i, W
    n_nodes, d = px.shape
    (n_edges,) = pair_j.shape
    fn = _make_gather(n_nodes, n_edges, d)
    return fn(px, pair_j.astype(jnp.int32))


# speedup vs baseline: 1.3790x; 1.0101x over previous
import functools

import jax
import jax.numpy as jnp
from jax import lax
from jax.experimental import pallas as pl
from jax.experimental.pallas import tpu as pltpu
from jax.experimental.pallas import tpu_sc as plsc


def _make_gather(n_nodes: int, n_edges: int, d: int):
    info = plsc.get_sparse_core_info()
    nc, ns = info.num_cores, info.num_subcores
    nw = nc * ns
    assert n_edges % nw == 0
    b_per_w = n_edges // nw
    chunk = 48
    nbuf = 6
    lag = 3
    n_chunks = b_per_w // chunk
    tail = b_per_w - n_chunks * chunk
    pre_rows = 104
    assert chunk % 8 == 0 and tail % 8 == 0 and b_per_w % 8 == 0

    mesh = plsc.VectorSubcoreMesh(core_axis_name="c", subcore_axis_name="s")

    @functools.partial(
        pl.kernel,
        mesh=mesh,
        out_type=jax.ShapeDtypeStruct((n_edges, d), jnp.float32),
        scratch_types=[
            pltpu.VMEM((b_per_w,), jnp.int32),
            pltpu.VMEM((nbuf * chunk, d), jnp.float32),
            pltpu.VMEM((tail, d), jnp.float32),
            pltpu.VMEM_SHARED((n_nodes, d), jnp.float32),
            pltpu.SemaphoreType.DMA((nbuf,)),
            pltpu.SemaphoreType.DMA((nbuf,)),
            pltpu.SemaphoreType.DMA,
            pltpu.SemaphoreType.DMA,
        ],
    )
    def gather_kernel(
        px_hbm, idx_hbm, out_hbm, idx_v, rows_v, tail_v, px_sh, gsem, ssem, tsem, stsem
    ):
        sid = lax.axis_index("s")
        wid = sid * nc + lax.axis_index("c")
        base = wid * b_per_w
        pltpu.async_copy(idx_hbm.at[pl.ds(base, b_per_w)], idx_v, tsem)

        rows_per_tile = 624
        n_pre = rows_per_tile // pre_rows
        for k in range(n_pre):
            off = sid * rows_per_tile + k * pre_rows
            pltpu.async_copy(
                px_hbm.at[pl.ds(off, pre_rows)], px_sh.at[pl.ds(off, pre_rows)], stsem
            )

        @pl.when(sid == 0)
        def _stage_rest():
            off = ns * rows_per_tile
            rest = n_nodes - ns * rows_per_tile
            pltpu.async_copy(px_hbm.at[pl.ds(off, rest)], px_sh.at[pl.ds(off, rest)], stsem)

        pltpu.make_async_copy(idx_hbm.at[pl.ds(0, b_per_w)], idx_v, tsem).wait()

        def start_gather(ci, b):
            idx_slice = idx_v.at[pl.ds(ci * chunk, chunk)]
            pltpu.async_copy(
                px_sh.at[idx_slice], rows_v.at[pl.ds(b * chunk, chunk)], gsem.at[b]
            )

        def wait_gather(b):
            pltpu.make_async_copy(
                px_sh.at[pl.ds(0, chunk)], rows_v.at[pl.ds(b * chunk, chunk)], gsem.at[b]
            ).wait()

        def start_store(ci, b):
            pltpu.async_copy(
                rows_v.at[pl.ds(b * chunk, chunk)],
                out_hbm.at[pl.ds(base + ci * chunk, chunk)],
                ssem.at[b],
            )

        def wait_store(b):
            pltpu.make_async_copy(
                rows_v.at[pl.ds(b * chunk, chunk)],
                out_hbm.at[pl.ds(0, chunk)],
                ssem.at[b],
            ).wait()

        def start_gather_hbm(ci, b):
            idx_slice = idx_v.at[pl.ds(ci * chunk, chunk)]
            pltpu.async_copy(
                px_hbm.at[idx_slice], rows_v.at[pl.ds(b * chunk, chunk)], gsem.at[b]
            )

        for i in range(lag):
            start_gather_hbm(i, i)
        for i in range(lag, nbuf):
            start_gather_hbm(i, i)
            wait_gather(i - lag)
            start_store(i - lag, i - lag)

        for k in range(n_pre):
            pltpu.make_async_copy(
                px_hbm.at[pl.ds(0, pre_rows)], px_sh.at[pl.ds(0, pre_rows)], stsem
            ).wait()

        @pl.when(sid == 0)
        def _drain_rest():
            rest = n_nodes - ns * rows_per_tile
            pltpu.make_async_copy(
                px_hbm.at[pl.ds(0, rest)], px_sh.at[pl.ds(0, rest)], stsem
            ).wait()

        plsc.subcore_barrier()

        pltpu.async_copy(
            px_sh.at[idx_v.at[pl.ds(n_chunks * chunk, tail)]], tail_v, tsem
        )

        def step(i, carry):
            bg = lax.rem(i, nbuf)
            wait_store(bg)
            start_gather(i, bg)
            j = i - lag
            bj = lax.rem(j, nbuf)
            wait_gather(bj)
            start_store(j, bj)
            return carry

        lax.fori_loop(nbuf, n_chunks, step, 0)

        for j in range(n_chunks - lag, n_chunks):
            wait_gather(j % nbuf)
            start_store(j, j % nbuf)

        pltpu.make_async_copy(px_sh.at[pl.ds(0, tail)], tail_v, tsem).wait()
        pltpu.async_copy(
            tail_v, out_hbm.at[pl.ds(base + n_chunks * chunk, tail)], tsem
        )

        for b in range(nbuf):
            wait_store(b)
        pltpu.make_async_copy(tail_v, out_hbm.at[pl.ds(0, tail)], tsem).wait()

    return gather_kernel


def kernel(px, pair_i, pair_j, W):
    del pair_i, W
    n_nodes, d = px.shape
    (n_edges,) = pair_j.shape
    fn = _make_gather(n_nodes, n_edges, d)
    return fn(px, pair_j.astype(jnp.int32))
